# baseline (speedup 1.0000x reference)
import functools

import jax
import jax.numpy as jnp
from jax import lax
from jax.experimental import pallas as pl
from jax.experimental.pallas import tpu as pltpu
from jax.experimental.pallas import tpu_sc as plsc

BATCH = 4096
DIM = 64
GRP = 8
INV_TEMP = 5.0

_NC = 2
_NS = 16
_NW = _NC * _NS
_BPW = BATCH // _NW
_CW = 16


def _lane(vec, j):
    return jnp.sum(jnp.where(lax.iota(jnp.int32, 16) == j, vec, 0))


def _sc_gather_body(uid_hbm, iid_hbm, utab_hbm, itab_hbm, uout_hbm, iout_hbm,
                    idx_v, sem_g):
    wid = lax.axis_index("s") * _NC + lax.axis_index("c")
    base = wid * _BPW
    for id_hbm, tab, out in ((uid_hbm, utab_hbm, uout_hbm),
                             (iid_hbm, itab_hbm, iout_hbm)):
        pltpu.sync_copy(id_hbm.at[pl.ds(base, _BPW)], idx_v)

        def chunk(c, _, tab=tab, out=out):
            vec = idx_v[pl.ds(c * _CW, _CW)]
            copies = []
            for j in range(_CW):
                g8 = pl.multiple_of(
                    lax.shift_left(
                        lax.shift_right_logical(_lane(vec, j), 3), 3), GRP)
                k8 = pl.multiple_of((base + c * _CW + j) * GRP, GRP)
                copies.append(pltpu.async_copy(
                    tab.at[pl.ds(g8, GRP)], out.at[pl.ds(k8, GRP)], sem_g))
            for cp in copies:
                cp.wait()
            return 0

        lax.fori_loop(0, _BPW // _CW, chunk, 0)


@jax.jit
def _sc_gather_groups(uid, iid, utab, itab):
    mesh = plsc.VectorSubcoreMesh(core_axis_name="c", subcore_axis_name="s")
    return pl.kernel(
        _sc_gather_body,
        mesh=mesh,
        out_type=[
            jax.ShapeDtypeStruct((BATCH * GRP, DIM), jnp.float32),
            jax.ShapeDtypeStruct((BATCH * GRP, DIM), jnp.float32),
        ],
        scratch_types=[
            pltpu.VMEM((_BPW,), jnp.int32),
            pltpu.SemaphoreType.DMA,
        ],
        compiler_params=pltpu.CompilerParams(use_tc_tiling_on_sc=True,
                                             needs_layout_passes=False),
    )(uid, iid, utab, itab)


def _select_norm_body(ug_ref, ig_ref, uoh_ref, ioh_ref, u_ref, i_ref):
    bm = uoh_ref.shape[0]
    for g_ref, oh_ref, o_ref in ((ug_ref, uoh_ref, u_ref),
                                 (ig_ref, ioh_ref, i_ref)):
        oh = oh_ref[...]
        x = g_ref[pl.Slice(0, bm, GRP), :] * oh[:, 0:1]
        for k in range(1, GRP):
            x = x + g_ref[pl.Slice(k, bm, GRP), :] * oh[:, k:k + 1]
        o_ref[...] = x / jnp.maximum(
            jnp.sqrt(jnp.sum(x * x, axis=1, keepdims=True)), 1e-12)


def _select_norm(ugroups, igroups, uoh, ioh, bm=512):
    grid = (BATCH // bm,)
    return pl.pallas_call(
        _select_norm_body,
        grid=grid,
        in_specs=[
            pl.BlockSpec((bm * GRP, DIM), lambda i: (i, 0)),
            pl.BlockSpec((bm * GRP, DIM), lambda i: (i, 0)),
            pl.BlockSpec((bm, GRP), lambda i: (i, 0)),
            pl.BlockSpec((bm, GRP), lambda i: (i, 0)),
        ],
        out_specs=[
            pl.BlockSpec((bm, DIM), lambda i: (i, 0)),
            pl.BlockSpec((bm, DIM), lambda i: (i, 0)),
        ],
        out_shape=[
            jax.ShapeDtypeStruct((BATCH, DIM), jnp.float32),
            jax.ShapeDtypeStruct((BATCH, DIM), jnp.float32),
        ],
        compiler_params=pltpu.CompilerParams(
            dimension_semantics=("parallel",)),
    )(ugroups, igroups, uoh, ioh)


def _mm_body(u_ref, v_ref, o_ref):
    un = u_ref[...]
    vn = v_ref[...]
    o_ref[...] = lax.dot_general(
        un, vn, (((1,), (1,)), ((), ())),
        preferred_element_type=jnp.float32) * INV_TEMP


def _logits(user_emb, item_emb, bm=512, bn=1024):
    grid = (BATCH // bm, BATCH // bn)
    return pl.pallas_call(
        _mm_body,
        grid=grid,
        in_specs=[
            pl.BlockSpec((bm, DIM), lambda i, j: (i, 0)),
            pl.BlockSpec((bn, DIM), lambda i, j: (j, 0)),
        ],
        out_specs=pl.BlockSpec((bm, bn), lambda i, j: (i, j)),
        out_shape=jax.ShapeDtypeStruct((BATCH, BATCH), jnp.float32),
        compiler_params=pltpu.CompilerParams(
            dimension_semantics=("parallel", "parallel")),
    )(user_emb, item_emb)


def kernel(user_ids, item_ids, user_table, item_table):
    uid = user_ids.astype(jnp.int32)
    iid = item_ids.astype(jnp.int32)
    uoh = jax.nn.one_hot(uid % GRP, GRP, dtype=jnp.float32)
    ioh = jax.nn.one_hot(iid % GRP, GRP, dtype=jnp.float32)
    ugroups, igroups = _sc_gather_groups(uid, iid, user_table, item_table)
    user_emb, item_emb = _select_norm(ugroups, igroups, uoh, ioh)
    return _logits(user_emb, item_emb)


# speedup vs baseline: 1.4983x; 1.4983x over previous
import functools

import jax
import jax.numpy as jnp
from jax import lax
from jax.experimental import pallas as pl
from jax.experimental.pallas import tpu as pltpu
from jax.experimental.pallas import tpu_sc as plsc

BATCH = 4096
DIM = 64
PAIR = 2 * DIM
INV_TEMP = 5.0

_NC = 2
_NS = 16
_NW = _NC * _NS
_BPW = BATCH // _NW


def _sc_gather_body(upix_hbm, ipix_hbm, utab_hbm, itab_hbm, uout_hbm, iout_hbm,
                    uidx_v, iidx_v, urows_v, irows_v, usem, isem):
    wid = lax.axis_index("s") * _NC + lax.axis_index("c")
    base = wid * _BPW
    pltpu.sync_copy(upix_hbm.at[pl.ds(base, _BPW)], uidx_v)
    pltpu.sync_copy(ipix_hbm.at[pl.ds(base, _BPW)], iidx_v)
    cu = pltpu.async_copy(utab_hbm.at[uidx_v], urows_v, usem)
    ci = pltpu.async_copy(itab_hbm.at[iidx_v], irows_v, isem)
    cu.wait()
    pltpu.sync_copy(urows_v, uout_hbm.at[pl.ds(base, _BPW)])
    ci.wait()
    pltpu.sync_copy(irows_v, iout_hbm.at[pl.ds(base, _BPW)])


@jax.jit
def _sc_gather_pairs(upix, ipix, utab2, itab2):
    mesh = plsc.VectorSubcoreMesh(core_axis_name="c", subcore_axis_name="s")
    return pl.kernel(
        _sc_gather_body,
        mesh=mesh,
        out_type=[
            jax.ShapeDtypeStruct((BATCH, PAIR), jnp.float32),
            jax.ShapeDtypeStruct((BATCH, PAIR), jnp.float32),
        ],
        scratch_types=[
            pltpu.VMEM((_BPW,), jnp.int32),
            pltpu.VMEM((_BPW,), jnp.int32),
            pltpu.VMEM((_BPW, PAIR), jnp.float32),
            pltpu.VMEM((_BPW, PAIR), jnp.float32),
            pltpu.SemaphoreType.DMA,
            pltpu.SemaphoreType.DMA,
        ],
        compiler_params=pltpu.CompilerParams(use_tc_tiling_on_sc=True),
    )(upix, ipix, utab2, itab2)


def _select_norm_body(up_ref, ip_ref, ub_ref, ib_ref, u_ref, i_ref):
    for p_ref, b_ref, o_ref in ((up_ref, ub_ref, u_ref),
                                (ip_ref, ib_ref, i_ref)):
        b = b_ref[...]
        x = p_ref[:, 0:DIM] * (1.0 - b) + p_ref[:, DIM:PAIR] * b
        o_ref[...] = x / jnp.maximum(
            jnp.sqrt(jnp.sum(x * x, axis=1, keepdims=True)), 1e-12)


def _select_norm(upairs, ipairs, ub, ib, bm=1024):
    grid = (BATCH // bm,)
    return pl.pallas_call(
        _select_norm_body,
        grid=grid,
        in_specs=[
            pl.BlockSpec((bm, PAIR), lambda i: (i, 0)),
            pl.BlockSpec((bm, PAIR), lambda i: (i, 0)),
            pl.BlockSpec((bm, 1), lambda i: (i, 0)),
            pl.BlockSpec((bm, 1), lambda i: (i, 0)),
        ],
        out_specs=[
            pl.BlockSpec((bm, DIM), lambda i: (i, 0)),
            pl.BlockSpec((bm, DIM), lambda i: (i, 0)),
        ],
        out_shape=[
            jax.ShapeDtypeStruct((BATCH, DIM), jnp.float32),
            jax.ShapeDtypeStruct((BATCH, DIM), jnp.float32),
        ],
        compiler_params=pltpu.CompilerParams(
            dimension_semantics=("parallel",)),
    )(upairs, ipairs, ub, ib)


def _mm_body(u_ref, v_ref, o_ref):
    un = u_ref[...]
    vn = v_ref[...]
    o_ref[...] = lax.dot_general(
        un, vn, (((1,), (1,)), ((), ())),
        preferred_element_type=jnp.float32) * INV_TEMP


def _logits(user_emb, item_emb, bm=512, bn=1024):
    grid = (BATCH // bm, BATCH // bn)
    return pl.pallas_call(
        _mm_body,
        grid=grid,
        in_specs=[
            pl.BlockSpec((bm, DIM), lambda i, j: (i, 0)),
            pl.BlockSpec((bn, DIM), lambda i, j: (j, 0)),
        ],
        out_specs=pl.BlockSpec((bm, bn), lambda i, j: (i, j)),
        out_shape=jax.ShapeDtypeStruct((BATCH, BATCH), jnp.float32),
        compiler_params=pltpu.CompilerParams(
            dimension_semantics=("parallel", "parallel")),
    )(user_emb, item_emb)


def kernel(user_ids, item_ids, user_table, item_table):
    uid = user_ids.astype(jnp.int32)
    iid = item_ids.astype(jnp.int32)
    upix = uid // 2
    ipix = iid // 2
    ub = (uid % 2).astype(jnp.float32)[:, None]
    ib = (iid % 2).astype(jnp.float32)[:, None]
    utab2 = user_table.reshape(-1, PAIR)
    itab2 = item_table.reshape(-1, PAIR)
    upairs, ipairs = _sc_gather_pairs(upix, ipix, utab2, itab2)
    user_emb, item_emb = _select_norm(upairs, ipairs, ub, ib)
    return _logits(user_emb, item_emb)


# speedup vs baseline: 3.2001x; 2.1358x over previous
import functools

import jax
import jax.numpy as jnp
from jax import lax
from jax.experimental import pallas as pl
from jax.experimental.pallas import tpu as pltpu
from jax.experimental.pallas import tpu_sc as plsc

BATCH = 4096
DIM = 64
GRP = 8
INV_TEMP = 5.0

_NC = 2
_NS = 16
_NW = _NC * _NS
_BPW = BATCH // _NW
_CW = 16


def _lane(vec, j):
    return jnp.sum(jnp.where(lax.iota(jnp.int32, 16) == j, vec, 0))


def _sc_gather_body(ugid_hbm, igid_hbm, utab_hbm, itab_hbm, uout_hbm, iout_hbm,
                    idx_v, buf_a, buf_b, sem_a, sem_b, sem_o):
    wid = lax.axis_index("s") * _NC + lax.axis_index("c")
    base = wid * _BPW
    nchunk = _BPW // _CW
    for gid_hbm, tab, out in ((ugid_hbm, utab_hbm, uout_hbm),
                              (igid_hbm, itab_hbm, iout_hbm)):
        pltpu.sync_copy(gid_hbm.at[pl.ds(base, _BPW)], idx_v)

        def fire(c, buf, sem, tab=tab):
            vec = idx_v[pl.ds(c * _CW, _CW)]
            copies = []
            for j in range(_CW):
                g = _lane(vec, j)
                copies.append(pltpu.async_copy(
                    tab.at[pl.ds(g, 1)], buf.at[pl.ds(j, 1)], sem))
            return copies

        def drain(c, copies, buf, out=out):
            for cp in copies:
                cp.wait()
            pltpu.async_copy(
                buf, out.at[pl.ds(base + c * _CW, _CW)], sem_o).wait()

        def two_chunks(h, _, tab=tab, out=out):
            ca = fire(2 * h, buf_a, sem_a)
            cb = fire(2 * h + 1, buf_b, sem_b)
            drain(2 * h, ca, buf_a)
            drain(2 * h + 1, cb, buf_b)
            return 0

        lax.fori_loop(0, nchunk // 2, two_chunks, 0)


@jax.jit
def _sc_gather_groups(ugid, igid, utab3, itab3):
    mesh = plsc.VectorSubcoreMesh(core_axis_name="c", subcore_axis_name="s")
    return pl.kernel(
        _sc_gather_body,
        mesh=mesh,
        out_type=[
            jax.ShapeDtypeStruct((BATCH, GRP, DIM), jnp.float32),
            jax.ShapeDtypeStruct((BATCH, GRP, DIM), jnp.float32),
        ],
        scratch_types=[
            pltpu.VMEM((_BPW,), jnp.int32),
            pltpu.VMEM((_CW, GRP, DIM), jnp.float32),
            pltpu.VMEM((_CW, GRP, DIM), jnp.float32),
            pltpu.SemaphoreType.DMA,
            pltpu.SemaphoreType.DMA,
            pltpu.SemaphoreType.DMA,
        ],
        compiler_params=pltpu.CompilerParams(use_tc_tiling_on_sc=True,
                                             needs_layout_passes=False),
    )(ugid, igid, utab3, itab3)


def _select_norm_body(ug_ref, ig_ref, uoh_ref, ioh_ref, u_ref, i_ref):
    for g_ref, oh_ref, o_ref in ((ug_ref, uoh_ref, u_ref),
                                 (ig_ref, ioh_ref, i_ref)):
        g = g_ref[...]
        oh = oh_ref[...]
        x = jnp.sum(g * oh[:, :, None], axis=1)
        o_ref[...] = x / jnp.maximum(
            jnp.sqrt(jnp.sum(x * x, axis=1, keepdims=True)), 1e-12)


def _select_norm(ugroups, igroups, uoh, ioh, bm=512):
    grid = (BATCH // bm,)
    return pl.pallas_call(
        _select_norm_body,
        grid=grid,
        in_specs=[
            pl.BlockSpec((bm, GRP, DIM), lambda i: (i, 0, 0)),
            pl.BlockSpec((bm, GRP, DIM), lambda i: (i, 0, 0)),
            pl.BlockSpec((bm, GRP), lambda i: (i, 0)),
            pl.BlockSpec((bm, GRP), lambda i: (i, 0)),
        ],
        out_specs=[
            pl.BlockSpec((bm, DIM), lambda i: (i, 0)),
            pl.BlockSpec((bm, DIM), lambda i: (i, 0)),
        ],
        out_shape=[
            jax.ShapeDtypeStruct((BATCH, DIM), jnp.float32),
            jax.ShapeDtypeStruct((BATCH, DIM), jnp.float32),
        ],
        compiler_params=pltpu.CompilerParams(
            dimension_semantics=("parallel",)),
    )(ugroups, igroups, uoh, ioh)


def _mm_body(u_ref, v_ref, o_ref):
    un = u_ref[...]
    vn = v_ref[...]
    o_ref[...] = lax.dot_general(
        un, vn, (((1,), (1,)), ((), ())),
        preferred_element_type=jnp.float32) * INV_TEMP


def _logits(user_emb, item_emb, bm=512, bn=1024):
    grid = (BATCH // bm, BATCH // bn)
    return pl.pallas_call(
        _mm_body,
        grid=grid,
        in_specs=[
            pl.BlockSpec((bm, DIM), lambda i, j: (i, 0)),
            pl.BlockSpec((bn, DIM), lambda i, j: (j, 0)),
        ],
        out_specs=pl.BlockSpec((bm, bn), lambda i, j: (i, j)),
        out_shape=jax.ShapeDtypeStruct((BATCH, BATCH), jnp.float32),
        compiler_params=pltpu.CompilerParams(
            dimension_semantics=("parallel", "parallel")),
    )(user_emb, item_emb)


def kernel(user_ids, item_ids, user_table, item_table):
    uid = user_ids.astype(jnp.int32)
    iid = item_ids.astype(jnp.int32)
    ugid = uid // GRP
    igid = iid // GRP
    uoh = jax.nn.one_hot(uid % GRP, GRP, dtype=jnp.float32)
    ioh = jax.nn.one_hot(iid % GRP, GRP, dtype=jnp.float32)
    utab3 = user_table.reshape(-1, GRP, DIM)
    itab3 = item_table.reshape(-1, GRP, DIM)
    ugroups, igroups = _sc_gather_groups(ugid, igid, utab3, itab3)
    user_emb, item_emb = _select_norm(ugroups, igroups, uoh, ioh)
    return _logits(user_emb, item_emb)


# speedup vs baseline: 3.2820x; 1.0256x over previous
import functools

import jax
import jax.numpy as jnp
from jax import lax
from jax.experimental import pallas as pl
from jax.experimental.pallas import tpu as pltpu
from jax.experimental.pallas import tpu_sc as plsc

BATCH = 4096
DIM = 64
GRP = 8
INV_TEMP = 5.0

_NC = 2
_NS = 16
_NW = _NC * _NS
_BPW = BATCH // _NW
_CW = 16


def _lane(vec, j):
    return jnp.sum(jnp.where(lax.iota(jnp.int32, 16) == j, vec, 0))


def _sc_gather_body(ugid_hbm, igid_hbm, utab_hbm, itab_hbm, uout_hbm, iout_hbm,
                    idx_v, buf_a, buf_b, sem_a, sem_b, sem_o):
    wid = lax.axis_index("s") * _NC + lax.axis_index("c")
    base = wid * _BPW
    nchunk = _BPW // _CW
    for gid_hbm, tab, out in ((ugid_hbm, utab_hbm, uout_hbm),
                              (igid_hbm, itab_hbm, iout_hbm)):
        pltpu.sync_copy(gid_hbm.at[pl.ds(base, _BPW)], idx_v)

        def fire(c, buf, sem, tab=tab):
            vec = idx_v[pl.ds(c * _CW, _CW)]
            copies = []
            for j in range(_CW):
                g = _lane(vec, j)
                copies.append(pltpu.async_copy(
                    tab.at[pl.ds(g, 1)], buf.at[pl.ds(j, 1)], sem))
            return copies

        def drain(c, copies, buf, out=out):
            for cp in copies:
                cp.wait()
            pltpu.async_copy(
                buf, out.at[pl.ds(base + c * _CW, _CW)], sem_o).wait()

        def two_chunks(h, _, tab=tab, out=out):
            ca = fire(2 * h, buf_a, sem_a)
            cb = fire(2 * h + 1, buf_b, sem_b)
            drain(2 * h, ca, buf_a)
            drain(2 * h + 1, cb, buf_b)
            return 0

        lax.fori_loop(0, nchunk // 2, two_chunks, 0)


@jax.jit
def _sc_gather_groups(ugid, igid, utab3, itab3):
    mesh = plsc.VectorSubcoreMesh(core_axis_name="c", subcore_axis_name="s")
    return pl.kernel(
        _sc_gather_body,
        mesh=mesh,
        out_type=[
            jax.ShapeDtypeStruct((BATCH, GRP, DIM), jnp.float32),
            jax.ShapeDtypeStruct((BATCH, GRP, DIM), jnp.float32),
        ],
        scratch_types=[
            pltpu.VMEM((_BPW,), jnp.int32),
            pltpu.VMEM((_CW, GRP, DIM), jnp.float32),
            pltpu.VMEM((_CW, GRP, DIM), jnp.float32),
            pltpu.SemaphoreType.DMA,
            pltpu.SemaphoreType.DMA,
            pltpu.SemaphoreType.DMA,
        ],
        compiler_params=pltpu.CompilerParams(use_tc_tiling_on_sc=True,
                                             needs_layout_passes=False),
    )(ugid, igid, utab3, itab3)


def _select_norm_body(ug_ref, ig_ref, uoh_ref, ioh_ref, u_ref, i_ref):
    for g_ref, oh_ref, o_ref in ((ug_ref, uoh_ref, u_ref),
                                 (ig_ref, ioh_ref, i_ref)):
        g = g_ref[...]
        oh = oh_ref[...]
        x = jnp.sum(g * oh[:, :, None], axis=1)
        o_ref[...] = x / jnp.maximum(
            jnp.sqrt(jnp.sum(x * x, axis=1, keepdims=True)), 1e-12)


def _select_norm(ugroups, igroups, uoh, ioh, bm=1024):
    grid = (BATCH // bm,)
    return pl.pallas_call(
        _select_norm_body,
        grid=grid,
        in_specs=[
            pl.BlockSpec((bm, GRP, DIM), lambda i: (i, 0, 0)),
            pl.BlockSpec((bm, GRP, DIM), lambda i: (i, 0, 0)),
            pl.BlockSpec((bm, GRP), lambda i: (i, 0)),
            pl.BlockSpec((bm, GRP), lambda i: (i, 0)),
        ],
        out_specs=[
            pl.BlockSpec((bm, DIM), lambda i: (i, 0)),
            pl.BlockSpec((bm, DIM), lambda i: (i, 0)),
        ],
        out_shape=[
            jax.ShapeDtypeStruct((BATCH, DIM), jnp.float32),
            jax.ShapeDtypeStruct((BATCH, DIM), jnp.float32),
        ],
        compiler_params=pltpu.CompilerParams(
            dimension_semantics=("parallel",)),
    )(ugroups, igroups, uoh, ioh)


def _mm_body(u_ref, v_ref, o_ref):
    un = u_ref[...]
    vn = v_ref[...]
    o_ref[...] = lax.dot_general(
        un, vn, (((1,), (1,)), ((), ())),
        preferred_element_type=jnp.float32) * INV_TEMP


def _logits(user_emb, item_emb, bm=1024, bn=2048):
    grid = (BATCH // bm, BATCH // bn)
    return pl.pallas_call(
        _mm_body,
        grid=grid,
        in_specs=[
            pl.BlockSpec((bm, DIM), lambda i, j: (i, 0)),
            pl.BlockSpec((bn, DIM), lambda i, j: (j, 0)),
        ],
        out_specs=pl.BlockSpec((bm, bn), lambda i, j: (i, j)),
        out_shape=jax.ShapeDtypeStruct((BATCH, BATCH), jnp.float32),
        compiler_params=pltpu.CompilerParams(
            dimension_semantics=("parallel", "parallel")),
    )(user_emb, item_emb)


def kernel(user_ids, item_ids, user_table, item_table):
    uid = user_ids.astype(jnp.int32)
    iid = item_ids.astype(jnp.int32)
    ugid = uid // GRP
    igid = iid // GRP
    uoh = jax.nn.one_hot(uid % GRP, GRP, dtype=jnp.float32)
    ioh = jax.nn.one_hot(iid % GRP, GRP, dtype=jnp.float32)
    utab3 = user_table.reshape(-1, GRP, DIM)
    itab3 = item_table.reshape(-1, GRP, DIM)
    ugroups, igroups = _sc_gather_groups(ugid, igid, utab3, itab3)
    user_emb, item_emb = _select_norm(ugroups, igroups, uoh, ioh)
    return _logits(user_emb, item_emb)


# speedup vs baseline: 3.2895x; 1.0023x over previous
import jax
import jax.numpy as jnp
from jax import lax
from jax.experimental import pallas as pl
from jax.experimental.pallas import tpu as pltpu
from jax.experimental.pallas import tpu_sc as plsc

BATCH = 4096
DIM = 64
GRP = 8
INV_TEMP = 5.0

_NC = 2
_NS = 16
_NW = _NC * _NS
_BPW = BATCH // _NW
_CW = 16


def _lane(vec, j):
    return jnp.sum(jnp.where(lax.iota(jnp.int32, 16) == j, vec, 0))


def _sc_gather_body(ugid_hbm, igid_hbm, utab_hbm, itab_hbm, uout_hbm, iout_hbm,
                    idx_v, buf_a, buf_b, sem_a, sem_b, sem_o):
    wid = lax.axis_index("s") * _NC + lax.axis_index("c")
    base = wid * _BPW
    nchunk = _BPW // _CW
    for gid_hbm, tab, out in ((ugid_hbm, utab_hbm, uout_hbm),
                              (igid_hbm, itab_hbm, iout_hbm)):
        pltpu.sync_copy(gid_hbm.at[pl.ds(base, _BPW)], idx_v)

        def fire(c, buf, sem, tab=tab):
            vec = idx_v[pl.ds(c * _CW, _CW)]
            copies = []
            for j in range(_CW):
                g = _lane(vec, j)
                copies.append(pltpu.async_copy(
                    tab.at[pl.ds(g, 1)], buf.at[pl.ds(j, 1)], sem))
            return copies

        def drain(c, copies, buf, out=out):
            for cp in copies:
                cp.wait()
            pltpu.async_copy(
                buf, out.at[pl.ds(base + c * _CW, _CW)], sem_o).wait()

        def two_chunks(h, _, tab=tab, out=out):
            ca = fire(2 * h, buf_a, sem_a)
            cb = fire(2 * h + 1, buf_b, sem_b)
            drain(2 * h, ca, buf_a)
            drain(2 * h + 1, cb, buf_b)
            return 0

        lax.fori_loop(0, nchunk // 2, two_chunks, 0)


@jax.jit
def _sc_gather_groups(ugid, igid, utab3, itab3):
    mesh = plsc.VectorSubcoreMesh(core_axis_name="c", subcore_axis_name="s")
    return pl.kernel(
        _sc_gather_body,
        mesh=mesh,
        out_type=[
            jax.ShapeDtypeStruct((BATCH, GRP, DIM), jnp.float32),
            jax.ShapeDtypeStruct((BATCH, GRP, DIM), jnp.float32),
        ],
        scratch_types=[
            pltpu.VMEM((_BPW,), jnp.int32),
            pltpu.VMEM((_CW, GRP, DIM), jnp.float32),
            pltpu.VMEM((_CW, GRP, DIM), jnp.float32),
            pltpu.SemaphoreType.DMA,
            pltpu.SemaphoreType.DMA,
            pltpu.SemaphoreType.DMA,
        ],
        compiler_params=pltpu.CompilerParams(use_tc_tiling_on_sc=True,
                                             needs_layout_passes=False),
    )(ugid, igid, utab3, itab3)


def _select_norm_body(ug_ref, ig_ref, uoh_ref, ioh_ref, u_ref, i_ref):
    for g_ref, oh_ref, o_ref in ((ug_ref, uoh_ref, u_ref),
                                 (ig_ref, ioh_ref, i_ref)):
        g = g_ref[...]
        oh = oh_ref[...]
        x = jnp.sum(g * oh[:, :, None], axis=1)
        o_ref[...] = x / jnp.maximum(
            jnp.sqrt(jnp.sum(x * x, axis=1, keepdims=True)), 1e-12)


def _select_norm(ugroups, igroups, uoh, ioh, bm=1024):
    grid = (BATCH // bm,)
    return pl.pallas_call(
        _select_norm_body,
        grid=grid,
        in_specs=[
            pl.BlockSpec((bm, GRP, DIM), lambda i: (i, 0, 0)),
            pl.BlockSpec((bm, GRP, DIM), lambda i: (i, 0, 0)),
            pl.BlockSpec((bm, GRP), lambda i: (i, 0)),
            pl.BlockSpec((bm, GRP), lambda i: (i, 0)),
        ],
        out_specs=[
            pl.BlockSpec((bm, DIM), lambda i: (i, 0)),
            pl.BlockSpec((bm, DIM), lambda i: (i, 0)),
        ],
        out_shape=[
            jax.ShapeDtypeStruct((BATCH, DIM), jnp.float32),
            jax.ShapeDtypeStruct((BATCH, DIM), jnp.float32),
        ],
        compiler_params=pltpu.CompilerParams(
            dimension_semantics=("parallel",)),
    )(ugroups, igroups, uoh, ioh)


def _mm_body(u_ref, v_ref, o_ref):
    un = u_ref[...]
    vn = v_ref[...]
    o_ref[...] = lax.dot_general(
        un, vn, (((1,), (1,)), ((), ())),
        preferred_element_type=jnp.float32) * INV_TEMP


def _logits(user_emb, item_emb, bm=1024, bn=2048):
    grid = (BATCH // bm, BATCH // bn)
    return pl.pallas_call(
        _mm_body,
        grid=grid,
        in_specs=[
            pl.BlockSpec((bm, DIM), lambda i, j: (i, 0)),
            pl.BlockSpec((bn, DIM), lambda i, j: (j, 0)),
        ],
        out_specs=pl.BlockSpec((bm, bn), lambda i, j: (i, j)),
        out_shape=jax.ShapeDtypeStruct((BATCH, BATCH), jnp.float32),
        compiler_params=pltpu.CompilerParams(
            dimension_semantics=("parallel", "parallel")),
    )(user_emb, item_emb)


def kernel(user_ids, item_ids, user_table, item_table):
    uid = user_ids.astype(jnp.int32)
    iid = item_ids.astype(jnp.int32)
    ugid = uid // GRP
    igid = iid // GRP
    uoh = jax.nn.one_hot(uid % GRP, GRP, dtype=jnp.float32)
    ioh = jax.nn.one_hot(iid % GRP, GRP, dtype=jnp.float32)
    utab3 = user_table.reshape(-1, GRP, DIM)
    itab3 = item_table.reshape(-1, GRP, DIM)
    ugroups, igroups = _sc_gather_groups(ugid, igid, utab3, itab3)
    user_emb, item_emb = _select_norm(ugroups, igroups, uoh, ioh)
    return _logits(user_emb, item_emb)
